# baseline (speedup 1.0000x reference)
import functools

import jax
import jax.numpy as jnp
from jax import lax
from jax.experimental import pallas as pl
from jax.experimental.pallas import tpu as pltpu
from jax.experimental.pallas import tpu_sc as plsc

_N = 10000
_D = 256
_E = 160000
_NPAD = 10112
_CH = 128
_EPAD = 163840
_CHE = 128
_NCHK = 80
_CHUNKS_C = 40
_RPT = _NPAD // 16


def _seg_sum(x0, x1, src, dst, zeros_g):
    mesh = plsc.VectorSubcoreMesh(core_axis_name="c", subcore_axis_name="s")

    @functools.partial(
        pl.kernel,
        out_type=(
            jax.ShapeDtypeStruct((_NPAD, 128), jnp.float32),
            jax.ShapeDtypeStruct((_NPAD, 128), jnp.float32),
        ),
        mesh=mesh,
        scratch_types=[
            pltpu.VMEM((_NCHK // 2, _CHE), jnp.int32),
            pltpu.VMEM((_NCHK // 2, _CHE), jnp.int32),
            pltpu.VMEM((_CHE, 128), jnp.float32),
            pltpu.VMEM((_CHE, 128), jnp.float32),
            pltpu.VMEM_SHARED((_NPAD, 128), jnp.float32),
            pltpu.SemaphoreType.DMA,
            pltpu.SemaphoreType.DMA,
        ],
    )
    def seg(x0_h, x1_h, src_h, dst_h, zg_h,
            g0_h, g1_h,
            src_v, dst_v, rows_a, rows_b, acc_s, sem_a, sem_b):
        c = lax.axis_index("c")
        s = lax.axis_index("s")
        base = s * _RPT
        half = _NCHK // 2

        pltpu.sync_copy(zg_h, acc_s.at[pl.ds(base, _RPT)])
        plsc.subcore_barrier()

        def pipeline(x_h):
            for h in range(2):
                pltpu.sync_copy(src_h.at[s, pl.ds(h * half, half)], src_v)
                pltpu.sync_copy(dst_h.at[s, pl.ds(h * half, half)], dst_v)
                pltpu.async_copy(x_h.at[src_v.at[0]], rows_a, sem_a)

                def body(jj, carry):
                    j0 = jj * 2
                    pltpu.async_copy(x_h.at[src_v.at[j0 + 1]], rows_b, sem_b)
                    pltpu.make_async_copy(x_h.at[src_v.at[j0]], rows_a, sem_a).wait()
                    pltpu.sync_copy(rows_a, acc_s.at[dst_v.at[j0]], add=True)

                    @pl.when(jj < half // 2 - 1)
                    def _():
                        pltpu.async_copy(x_h.at[src_v.at[j0 + 2]], rows_a, sem_a)

                    pltpu.make_async_copy(x_h.at[src_v.at[j0 + 1]], rows_b, sem_b).wait()
                    pltpu.sync_copy(rows_b, acc_s.at[dst_v.at[j0 + 1]], add=True)
                    return carry

                lax.fori_loop(0, half // 2, body, 0)

        @pl.when(c == 0)
        def _():
            pipeline(x0_h)

        @pl.when(c == 1)
        def _():
            pipeline(x1_h)

        plsc.subcore_barrier()

        @pl.when(c == 0)
        def _():
            pltpu.sync_copy(acc_s.at[pl.ds(base, _RPT)], g0_h.at[pl.ds(base, _RPT)])

        @pl.when(c == 1)
        def _():
            pltpu.sync_copy(acc_s.at[pl.ds(base, _RPT)], g1_h.at[pl.ds(base, _RPT)])

    return seg(x0, x1, src, dst, zeros_g)


def _seg_cnt(dsts, zeros_g, ones_h):
    mesh = plsc.VectorSubcoreMesh(core_axis_name="c", subcore_axis_name="s")

    @functools.partial(
        pl.kernel,
        out_type=tuple(
            jax.ShapeDtypeStruct((_NPAD, 128), jnp.float32) for _ in range(6)
        ),
        mesh=mesh,
        scratch_types=[
            pltpu.VMEM((_CHUNKS_C, _CH), jnp.int32),
            pltpu.VMEM((_CH, 128), jnp.float32),
            pltpu.VMEM_SHARED((_NPAD, 128), jnp.float32),
        ],
    )
    def cntk(d0_h, d1_h, d2_h, zg_h, on_h,
             o00, o01, o10, o11, o20, o21,
             dst_v, ones_v, acc_s):
        c = lax.axis_index("c")
        s = lax.axis_index("s")
        wid = s * 2 + c
        base = s * _RPT

        pltpu.sync_copy(on_h, ones_v)
        outs = ((d0_h, o00, o01), (d1_h, o10, o11), (d2_h, o20, o21))
        for d_h, out0, out1 in outs:
            pltpu.sync_copy(zg_h, acc_s.at[pl.ds(base, _RPT)])
            pltpu.sync_copy(d_h.at[wid], dst_v)
            plsc.subcore_barrier()

            def body(j, carry):
                pltpu.sync_copy(ones_v, acc_s.at[dst_v.at[j]], add=True)
                return carry

            lax.fori_loop(0, _CHUNKS_C, body, 0)
            plsc.subcore_barrier()

            @pl.when(c == 0)
            def _():
                pltpu.sync_copy(acc_s.at[pl.ds(base, _RPT)], out0.at[pl.ds(base, _RPT)])

            @pl.when(c == 1)
            def _():
                pltpu.sync_copy(acc_s.at[pl.ds(base, _RPT)], out1.at[pl.ds(base, _RPT)])

            plsc.subcore_barrier()

    return cntk(*dsts, zeros_g, ones_h)


def _prep_edges(ei):
    src = jnp.asarray(ei[0], jnp.int32)
    dst = jnp.asarray(ei[1], jnp.int32)
    pad = _EPAD - _E
    src = jnp.concatenate([src, jnp.zeros((pad,), jnp.int32)])
    trash = _N + jnp.arange(pad, dtype=jnp.int32) % (_NPAD - _N)
    dst = jnp.concatenate([dst, trash])
    return (src.reshape(16, _NCHK, _CHE), dst.reshape(16, _NCHK, _CHE),
            dst.reshape(32, _CHUNKS_C, _CH))


_R = 1000


def _tc_body(x_ref, g0d, g1d, g0l, g1l, g0g, g1g,
             cd0, cd1, cl0, cl1, cg0, cg1,
             w2, w1dl, w1dh, w1ll, w1lh, w1gl, w1gh, wu, bm, out):
    xb = x_ref[...]
    prec = lax.Precision.DEFAULT
    p = jnp.dot(xb, w2[...], precision=prec, preferred_element_type=jnp.float32)
    acc = jnp.dot(xb, wu[0:_D, :], precision=prec, preferred_element_type=jnp.float32)
    acc += bm[3:4, :]
    parts = ((g0d, g1d, cd0, cd1, w1dl, w1dh), (g0l, g1l, cl0, cl1, w1ll, w1lh),
             (g0g, g1g, cg0, cg1, w1gl, w1gh))
    for t, (g0, g1, c0, c1, w1l, w1h) in enumerate(parts):
        q = (jnp.dot(g0[...], w1l[...], precision=prec, preferred_element_type=jnp.float32)
             + jnp.dot(g1[...], w1h[...], precision=prec, preferred_element_type=jnp.float32))
        cv = c0[:, 0:1] + c1[:, 0:1]
        inv = 1.0 / jnp.maximum(cv, 1.0)
        msk = jnp.minimum(cv, 1.0)
        m = q * inv + msk * (p[:, t * _D:(t + 1) * _D] + bm[t:t + 1, :])
        acc += jnp.dot(m, wu[(t + 1) * _D:(t + 2) * _D, :], precision=prec,
                       preferred_element_type=jnp.float32)
    out[...] = jnp.maximum(acc, 0.0)


def _update(x, gs, cnts, w2cat, w1halves, wu, bmat):
    grid = (_N // _R,)
    row = lambda i: (i, 0)
    fixed = lambda i: (0, 0)
    in_specs = (
        [pl.BlockSpec((_R, _D), row)]
        + [pl.BlockSpec((_R, 128), row)] * 6
        + [pl.BlockSpec((_R, 128), row)] * 6
        + [pl.BlockSpec((_D, 3 * _D), fixed)]
        + [pl.BlockSpec((128, _D), fixed)] * 6
        + [pl.BlockSpec((4 * _D, _D), fixed)]
        + [pl.BlockSpec((8, _D), fixed)]
    )
    return pl.pallas_call(
        _tc_body,
        grid=grid,
        in_specs=in_specs,
        out_specs=pl.BlockSpec((_R, _D), row),
        out_shape=jax.ShapeDtypeStruct((_N, _D), jnp.float32),
    )(x, *gs, *cnts, w2cat, *w1halves, wu, bmat)


def kernel(x, edge_index_diff, edge_index_lt, edge_index_gt,
           W_diff, b_diff, W_lt, b_lt, W_gt, b_gt, W_upd, b_upd):
    x = jnp.asarray(x, jnp.float32)
    x0 = x[:, :128]
    x1 = x[:, 128:]
    zeros_g = jnp.zeros((_RPT, 128), jnp.float32)
    ones_h = jnp.ones((_CH, 128), jnp.float32)

    gs, dsts_c = [], []
    for ei in (edge_index_diff, edge_index_lt, edge_index_gt):
        src_g, dst_g, dst_c = _prep_edges(ei)
        g0, g1 = _seg_sum(x0, x1, src_g, dst_g, zeros_g)
        gs += [g0, g1]
        dsts_c.append(dst_c)
    cnts = _seg_cnt(dsts_c, zeros_g, ones_h)

    w2cat = jnp.concatenate([W_diff[_D:], W_lt[_D:], W_gt[_D:]], axis=1)
    w1halves = [W_diff[:128], W_diff[128:_D], W_lt[:128], W_lt[128:_D],
                W_gt[:128], W_gt[128:_D]]
    bmat = jnp.zeros((8, _D), jnp.float32)
    bmat = bmat.at[0].set(b_diff).at[1].set(b_lt).at[2].set(b_gt).at[3].set(b_upd)
    return _update(x, gs, list(cnts), w2cat, w1halves, W_upd, bmat)


# speedup vs baseline: 1.1623x; 1.1623x over previous
import functools

import jax
import jax.numpy as jnp
from jax import lax
from jax.experimental import pallas as pl
from jax.experimental.pallas import tpu as pltpu
from jax.experimental.pallas import tpu_sc as plsc

_N = 10000
_D = 256
_E = 160000
_NPAD = 10112
_CH = 128
_EPAD = 163840
_CHE = 128
_NCHK = 80
_CHUNKS_C = 40
_RPT = _NPAD // 16


def _seg_all(x0, x1, srcs, dsts, dsts_c, zeros_g, ones_h):
    mesh = plsc.VectorSubcoreMesh(core_axis_name="c", subcore_axis_name="s")

    @functools.partial(
        pl.kernel,
        out_type=tuple(
            jax.ShapeDtypeStruct((_NPAD, 128), jnp.float32) for _ in range(12)
        ),
        mesh=mesh,
        scratch_types=[
            pltpu.VMEM((_NCHK // 2, _CHE), jnp.int32),
            pltpu.VMEM((_NCHK // 2, _CHE), jnp.int32),
            pltpu.VMEM((_CHE, 128), jnp.float32),
            pltpu.VMEM((_CHE, 128), jnp.float32),
            pltpu.VMEM_SHARED((_NPAD, 128), jnp.float32),
            pltpu.SemaphoreType.DMA,
            pltpu.SemaphoreType.DMA,
        ],
    )
    def seg(x0_h, x1_h, s0_h, d0_h, s1_h, d1_h, s2_h, d2_h,
            dc0_h, dc1_h, dc2_h, zg_h, on_h,
            g0d, g1d, g0l, g1l, g0g, g1g, c0d, c1d, c0l, c1l, c0g, c1g,
            src_v, dst_v, rows_a, rows_b, acc_s, sem_a, sem_b):
        c = lax.axis_index("c")
        s = lax.axis_index("s")
        wid = s * 2 + c
        base = s * _RPT
        slab = pl.ds(base, _RPT)
        half = _NCHK // 2

        pltpu.sync_copy(zg_h, acc_s.at[slab])
        plsc.subcore_barrier()

        def pipeline(x_h, src_h, dst_h):
            for h in range(2):
                pltpu.sync_copy(src_h.at[s, pl.ds(h * half, half)], src_v)
                pltpu.sync_copy(dst_h.at[s, pl.ds(h * half, half)], dst_v)
                pltpu.async_copy(x_h.at[src_v.at[0]], rows_a, sem_a)

                def body(jj, carry):
                    j0 = jj * 2
                    pltpu.async_copy(x_h.at[src_v.at[j0 + 1]], rows_b, sem_b)
                    pltpu.make_async_copy(x_h.at[src_v.at[j0]], rows_a, sem_a).wait()
                    pltpu.sync_copy(rows_a, acc_s.at[dst_v.at[j0]], add=True)

                    @pl.when(jj < half // 2 - 1)
                    def _():
                        pltpu.async_copy(x_h.at[src_v.at[j0 + 2]], rows_a, sem_a)

                    pltpu.make_async_copy(x_h.at[src_v.at[j0 + 1]], rows_b, sem_b).wait()
                    pltpu.sync_copy(rows_b, acc_s.at[dst_v.at[j0 + 1]], add=True)
                    return carry

                lax.fori_loop(0, half // 2, body, 0)

        for src_h, dst_h, o0, o1 in ((s0_h, d0_h, g0d, g1d),
                                     (s1_h, d1_h, g0l, g1l),
                                     (s2_h, d2_h, g0g, g1g)):
            @pl.when(c == 0)
            def _():
                pipeline(x0_h, src_h, dst_h)

            @pl.when(c == 1)
            def _():
                pipeline(x1_h, src_h, dst_h)

            plsc.subcore_barrier()

            @pl.when(c == 0)
            def _():
                pltpu.sync_copy(acc_s.at[slab], o0.at[slab])

            @pl.when(c == 1)
            def _():
                pltpu.sync_copy(acc_s.at[slab], o1.at[slab])

            pltpu.sync_copy(zg_h, acc_s.at[slab])
            plsc.subcore_barrier()

        pltpu.sync_copy(on_h, rows_a)
        for dc_h, o0, o1 in ((dc0_h, c0d, c1d), (dc1_h, c0l, c1l),
                             (dc2_h, c0g, c1g)):
            pltpu.sync_copy(dc_h.at[wid], dst_v)

            def cbody(j, carry):
                pltpu.sync_copy(rows_a, acc_s.at[dst_v.at[j]], add=True)
                return carry

            lax.fori_loop(0, _CHUNKS_C, cbody, 0)
            plsc.subcore_barrier()

            @pl.when(c == 0)
            def _():
                pltpu.sync_copy(acc_s.at[slab], o0.at[slab])

            @pl.when(c == 1)
            def _():
                pltpu.sync_copy(acc_s.at[slab], o1.at[slab])

            pltpu.sync_copy(zg_h, acc_s.at[slab])
            plsc.subcore_barrier()

    return seg(x0, x1, srcs[0], dsts[0], srcs[1], dsts[1], srcs[2], dsts[2],
               *dsts_c, zeros_g, ones_h)


def _prep_edges(ei):
    src = jnp.asarray(ei[0], jnp.int32)
    dst = jnp.asarray(ei[1], jnp.int32)
    pad = _EPAD - _E
    src = jnp.concatenate([src, jnp.zeros((pad,), jnp.int32)])
    ar = jnp.arange(pad, dtype=jnp.int32)
    dst_g = jnp.concatenate([dst, _N + ar % (_NPAD - _N)])
    dst_c = jnp.concatenate([dst, _N + (pad - 1 - ar) % (_NPAD - _N)])
    return (src.reshape(16, _NCHK, _CHE), dst_g.reshape(16, _NCHK, _CHE),
            dst_c.reshape(32, _CHUNKS_C, _CH))


_R = 1000


def _tc_body(x_ref, g0d, g1d, g0l, g1l, g0g, g1g,
             cd0, cd1, cl0, cl1, cg0, cg1,
             w2, w1dl, w1dh, w1ll, w1lh, w1gl, w1gh, wu, bm, out):
    xb = x_ref[...]
    prec = lax.Precision.DEFAULT
    p = jnp.dot(xb, w2[...], precision=prec, preferred_element_type=jnp.float32)
    acc = jnp.dot(xb, wu[0:_D, :], precision=prec, preferred_element_type=jnp.float32)
    acc += bm[3:4, :]
    parts = ((g0d, g1d, cd0, cd1, w1dl, w1dh), (g0l, g1l, cl0, cl1, w1ll, w1lh),
             (g0g, g1g, cg0, cg1, w1gl, w1gh))
    for t, (g0, g1, c0, c1, w1l, w1h) in enumerate(parts):
        q = (jnp.dot(g0[...], w1l[...], precision=prec, preferred_element_type=jnp.float32)
             + jnp.dot(g1[...], w1h[...], precision=prec, preferred_element_type=jnp.float32))
        cv = c0[:, 0:1] + c1[:, 0:1]
        inv = 1.0 / jnp.maximum(cv, 1.0)
        msk = jnp.minimum(cv, 1.0)
        m = q * inv + msk * (p[:, t * _D:(t + 1) * _D] + bm[t:t + 1, :])
        acc += jnp.dot(m, wu[(t + 1) * _D:(t + 2) * _D, :], precision=prec,
                       preferred_element_type=jnp.float32)
    out[...] = jnp.maximum(acc, 0.0)


def _update(x, gs, cnts, w2cat, w1halves, wu, bmat):
    grid = (_N // _R,)
    row = lambda i: (i, 0)
    fixed = lambda i: (0, 0)
    in_specs = (
        [pl.BlockSpec((_R, _D), row)]
        + [pl.BlockSpec((_R, 128), row)] * 6
        + [pl.BlockSpec((_R, 128), row)] * 6
        + [pl.BlockSpec((_D, 3 * _D), fixed)]
        + [pl.BlockSpec((128, _D), fixed)] * 6
        + [pl.BlockSpec((4 * _D, _D), fixed)]
        + [pl.BlockSpec((8, _D), fixed)]
    )
    return pl.pallas_call(
        _tc_body,
        grid=grid,
        in_specs=in_specs,
        out_specs=pl.BlockSpec((_R, _D), row),
        out_shape=jax.ShapeDtypeStruct((_N, _D), jnp.float32),
    )(x, *gs, *cnts, w2cat, *w1halves, wu, bmat)


def kernel(x, edge_index_diff, edge_index_lt, edge_index_gt,
           W_diff, b_diff, W_lt, b_lt, W_gt, b_gt, W_upd, b_upd):
    x = jnp.asarray(x, jnp.float32)
    x0 = x[:, :128]
    x1 = x[:, 128:]
    zeros_g = jnp.zeros((_RPT, 128), jnp.float32)
    ones_h = jnp.ones((_CH, 128), jnp.float32)

    srcs, dsts, dsts_c = [], [], []
    for ei in (edge_index_diff, edge_index_lt, edge_index_gt):
        src_g, dst_g, dst_c = _prep_edges(ei)
        srcs.append(src_g)
        dsts.append(dst_g)
        dsts_c.append(dst_c)
    outs = _seg_all(x0, x1, srcs, dsts, dsts_c, zeros_g, ones_h)
    gs, cnts = list(outs[:6]), list(outs[6:])

    w2cat = jnp.concatenate([W_diff[_D:], W_lt[_D:], W_gt[_D:]], axis=1)
    w1halves = [W_diff[:128], W_diff[128:_D], W_lt[:128], W_lt[128:_D],
                W_gt[:128], W_gt[128:_D]]
    bmat = jnp.zeros((8, _D), jnp.float32)
    bmat = bmat.at[0].set(b_diff).at[1].set(b_lt).at[2].set(b_gt).at[3].set(b_upd)
    return _update(x, gs, list(cnts), w2cat, w1halves, W_upd, bmat)
